# baseline (speedup 1.0000x reference)
import functools

import jax
import jax.numpy as jnp
from jax import lax
from jax.experimental import pallas as pl
from jax.experimental.pallas import tpu as pltpu
from jax.experimental.pallas import tpu_sc as plsc

_NC = 2
_NS = 16
_NW = _NC * _NS
_R = 128
_NBUF = 4
_K = 2


def _ring_copy(src, dst, base, rows, vb, fsem, dsem):
    nb = rows // _R
    fills = [
        pltpu.make_async_copy(
            src.at[pl.ds(base + b * _R, _R)], vb.at[b % _NBUF], fsem.at[b % _NBUF]
        )
        for b in range(nb)
    ]
    drains = [
        pltpu.make_async_copy(
            vb.at[b % _NBUF], dst.at[pl.ds(base + b * _R, _R)], dsem.at[b % _NBUF]
        )
        for b in range(nb)
    ]
    for b in range(min(_NBUF, nb)):
        fills[b].start()
    waited = -1
    for b in range(nb):
        fills[b].wait()
        drains[b].start()
        j = b - _K
        if j >= 0 and j + _NBUF < nb:
            drains[j].wait()
            fills[j + _NBUF].start()
            waited = j
    for b in range(waited + 1, nb):
        drains[b].wait()


def _sc_body(emb, lab, eq, lq, out_eq, out_lq, vb, vl, fsem, dsem, lfsem, ldsem):
    B = emb.shape[0]
    S = eq.shape[0]
    rows_pw = S // _NW
    nw_emb = B // rows_pw
    wid = lax.axis_index("s") * _NC + lax.axis_index("c")
    base = wid * rows_pw

    @pl.when(wid < nw_emb)
    def _():
        _ring_copy(emb, out_eq, base, rows_pw, vb, fsem, dsem)
        cf = pltpu.make_async_copy(lab.at[pl.ds(base, rows_pw)], vl, lfsem)
        cf.start()
        cf.wait()
        cd = pltpu.make_async_copy(vl, out_lq.at[pl.ds(base, rows_pw)], ldsem)
        cd.start()
        cd.wait()

    @pl.when(wid >= nw_emb)
    def _():
        _ring_copy(eq, out_eq, base, rows_pw, vb, fsem, dsem)
        cf = pltpu.make_async_copy(lq.at[pl.ds(base, rows_pw)], vl, lfsem)
        cf.start()
        cf.wait()
        cd = pltpu.make_async_copy(vl, out_lq.at[pl.ds(base, rows_pw)], ldsem)
        cd.start()
        cd.wait()


def kernel(embeddings, labels, embed_queue, label_queue):
    B, D = embeddings.shape
    S = embed_queue.shape[0]
    rows_pw = S // _NW
    mesh = plsc.VectorSubcoreMesh(core_axis_name="c", subcore_axis_name="s")
    sc_call = functools.partial(
        pl.kernel,
        mesh=mesh,
        out_type=[
            jax.ShapeDtypeStruct(embed_queue.shape, embed_queue.dtype),
            jax.ShapeDtypeStruct(label_queue.shape, label_queue.dtype),
        ],
        scratch_types=[
            pltpu.VMEM((_NBUF, _R, D), embed_queue.dtype),
            pltpu.VMEM((rows_pw,), label_queue.dtype),
            pltpu.SemaphoreType.DMA((_NBUF,)),
            pltpu.SemaphoreType.DMA((_NBUF,)),
            pltpu.SemaphoreType.DMA,
            pltpu.SemaphoreType.DMA,
        ],
    )
    out_eq, out_lq = sc_call(_sc_body)(embeddings, labels, embed_queue, label_queue)
    new_ptr = jnp.array([B % S], dtype=jnp.int32)
    return out_eq, out_lq, new_ptr
